# baseline (speedup 1.0000x reference)
import functools

import jax
import jax.numpy as jnp
from jax import lax
from jax.experimental import pallas as pl
from jax.experimental.pallas import tpu as pltpu
from jax.experimental.pallas import tpu_sc as plsc

E_DIM = 32
NUM_ASP = 18


@functools.cache
def _make_gather(B: int):
    info = plsc.get_sparse_core_info()
    NC, NS = info.num_cores, info.num_subcores
    NW = NC * NS
    assert B % (8 * NW) == 0
    bpw = B // NW

    mesh = plsc.VectorSubcoreMesh(core_axis_name="c", subcore_axis_name="s")
    f32 = jnp.float32

    @functools.partial(
        pl.kernel,
        mesh=mesh,
        compiler_params=pltpu.CompilerParams(use_tc_tiling_on_sc=False),
        out_type=(
            jax.ShapeDtypeStruct((B, E_DIM), f32),
            jax.ShapeDtypeStruct((B, E_DIM), f32),
            jax.ShapeDtypeStruct((B,), f32),
            jax.ShapeDtypeStruct((B,), f32),
        ),
        scratch_types=[
            pltpu.VMEM((bpw,), jnp.int32),
            pltpu.VMEM((bpw,), jnp.int32),
            pltpu.VMEM((bpw, E_DIM), f32),
            pltpu.VMEM((bpw, E_DIM), f32),
            pltpu.VMEM((bpw,), f32),
            pltpu.VMEM((bpw,), f32),
            pltpu.SemaphoreType.DMA,
            pltpu.SemaphoreType.DMA,
            pltpu.SemaphoreType.DMA,
            pltpu.SemaphoreType.DMA,
        ],
    )
    def gather(x_hbm, y_hbm, ut_hbm, it_hbm, ubt_hbm, ibt_hbm,
               urow_out, irow_out, ub_out, ib_out,
               xv, yv, uv, iv, ubv, ibv, s0, s1, s2, s3):
        wid = lax.axis_index("s") * NC + lax.axis_index("c")
        base = wid * bpw
        pltpu.sync_copy(x_hbm.at[pl.ds(base, bpw)], xv)
        pltpu.sync_copy(y_hbm.at[pl.ds(base, bpw)], yv)
        c0 = pltpu.async_copy(ut_hbm.at[xv], uv, s0)
        c1 = pltpu.async_copy(it_hbm.at[yv], iv, s1)
        c2 = pltpu.async_copy(ubt_hbm.at[xv], ubv, s2)
        c3 = pltpu.async_copy(ibt_hbm.at[yv], ibv, s3)
        c0.wait()
        pltpu.sync_copy(uv, urow_out.at[pl.ds(base, bpw)])
        c1.wait()
        pltpu.sync_copy(iv, irow_out.at[pl.ds(base, bpw)])
        c2.wait()
        pltpu.sync_copy(ubv, ub_out.at[pl.ds(base, bpw)])
        c3.wait()
        pltpu.sync_copy(ibv, ib_out.at[pl.ds(base, bpw)])

    return gather


def _dot_t(a, b):
    return lax.dot_general(a, b, (((1,), (1,)), ((), ())),
                           preferred_element_type=jnp.float32)


def _dense_body(u_ref, i_ref, ub_ref, ib_ref, asp_ref, aw_ref,
                w1_ref, b1_ref, w2_ref, b2_ref, w3_ref, b3_ref,
                out_ref, sim_ref, pref_ref):
    u = u_ref[...]
    it = i_ref[...]
    aw = aw_ref[...]
    out_ref[...] = (jnp.sum(u * it, axis=-1, keepdims=True)
                    + ub_ref[...] + ib_ref[...] + 3.53)
    wa = jnp.sqrt(jnp.sum(aw * aw, axis=1))
    aspv = asp_ref[...]
    t = aspv / jnp.maximum(jnp.abs(aspv) * wa[None, :], 1e-12)
    h = _dot_t(it, w1_ref[...]) + b1_ref[...]
    h = _dot_t(h, w2_ref[...]) + b2_ref[...]
    logits = _dot_t(h, w3_ref[...]) + b3_ref[...]
    weight = 1.0 / (1.0 + jnp.exp(-logits))
    item_asp = lax.dot_general(t * weight, aw, (((1,), (0,)), ((), ())),
                               preferred_element_type=jnp.float32)
    d = item_asp - it + 1e-6
    sim_ref[...] = jnp.sqrt(jnp.sum(d * d, axis=-1, keepdims=True))
    pref_ref[...] = t * _dot_t(u, aw)


def _dense(u_rows, i_rows, ub, ib, asp, asp_W, W1, b1, W2, b2, W3, b3):
    B = u_rows.shape[0]
    BLK = 2048
    grid = (B // BLK,)
    f32 = jnp.float32
    row = lambda b: (b, 0)
    rep = lambda b: (0, 0)
    return pl.pallas_call(
        _dense_body,
        grid=grid,
        in_specs=[
            pl.BlockSpec((BLK, E_DIM), row),
            pl.BlockSpec((BLK, E_DIM), row),
            pl.BlockSpec((BLK, 1), row),
            pl.BlockSpec((BLK, 1), row),
            pl.BlockSpec((BLK, NUM_ASP), row),
            pl.BlockSpec((NUM_ASP, E_DIM), rep),
            pl.BlockSpec((50, E_DIM), rep),
            pl.BlockSpec((1, 50), rep),
            pl.BlockSpec((25, 50), rep),
            pl.BlockSpec((1, 25), rep),
            pl.BlockSpec((NUM_ASP, 25), rep),
            pl.BlockSpec((1, NUM_ASP), rep),
        ],
        out_specs=[
            pl.BlockSpec((BLK, 1), row),
            pl.BlockSpec((BLK, 1), row),
            pl.BlockSpec((BLK, NUM_ASP), row),
        ],
        out_shape=[
            jax.ShapeDtypeStruct((B, 1), f32),
            jax.ShapeDtypeStruct((B, 1), f32),
            jax.ShapeDtypeStruct((B, NUM_ASP), f32),
        ],
    )(u_rows, i_rows, ub, ib, asp, asp_W,
      W1, b1.reshape(1, 50), W2, b2.reshape(1, 25), W3, b3.reshape(1, NUM_ASP))


def kernel(x, y, asp, user_table, item_table, u_bias, i_bias, asp_W,
           W1, b1, W2, b2, W3, b3):
    B = x.shape[0]
    x = x.astype(jnp.int32)
    y = y.astype(jnp.int32)
    u_rows, i_rows, ub, ib = _make_gather(B)(
        x, y, user_table, item_table, u_bias, i_bias)
    out2, sim2, pref = _dense(u_rows, i_rows, ub.reshape(B, 1),
                              ib.reshape(B, 1), asp, asp_W,
                              W1, b1, W2, b2, W3, b3)
    return out2.reshape(B), sim2.reshape(B), pref
